# baseline (speedup 1.0000x reference)
import jax
import jax.numpy as jnp
from jax import lax
from jax.experimental import pallas as pl
from jax.experimental.pallas import tpu as pltpu
from jax.experimental.pallas import tpu_sc as plsc

N = 10000
C = 128
K = 27
EPS = 1e-5
JB = 128
NPAD = 10240
NBLK = NPAD // JB
NW = 32
TPAD = N + 16
ZROW = N
NBUF = 4


def _sc_gather_body(feats_hbm, nbr_hbm, g_hbm):
    def body(i_vmem, o_vmem):
        @pl.loop(0, JB, step=16)
        def _(c):
            v = i_vmem[pl.ds(c, 16)]
            i_vmem[pl.ds(c, 16)] = jnp.where(v >= 0, v, ZROW)

        pltpu.sync_copy(feats_hbm.at[i_vmem], o_vmem)

    pltpu.emit_pipeline(
        body,
        grid=(K * NBLK,),
        in_specs=[pl.BlockSpec((JB,), lambda i: (i,))],
        out_specs=[pl.BlockSpec((JB, C), lambda i: (i, 0))],
        core_axis_name=("c", "s"),
        dimension_semantics=(pltpu.PARALLEL,),
    )(nbr_hbm, g_hbm)


def _sc_gather(feats_pad, nbr_flat):
    mesh = plsc.VectorSubcoreMesh(core_axis_name="c", subcore_axis_name="s")
    f = pl.kernel(
        _sc_gather_body,
        out_type=jax.ShapeDtypeStruct((K * NPAD, C), jnp.float32),
        mesh=mesh,
    )
    return f(feats_pad, nbr_flat)


def _tc_body(g_ref, w_ref, gamma_ref, beta_ref, o_ref, acc_ref):
    k = pl.program_id(0)

    @pl.when(k == 0)
    def _():
        acc_ref[...] = jnp.zeros_like(acc_ref)

    acc_ref[...] += lax.dot_general(
        g_ref[0], w_ref[0], (((1,), (0,)), ((), ())),
        preferred_element_type=jnp.float32)

    @pl.when(k == K - 1)
    def _():
        x = acc_ref[...]
        row = lax.broadcasted_iota(jnp.int32, (NPAD, 1), 0)
        m = (row < N).astype(jnp.float32)
        xm = x * m
        mean = jnp.sum(xm, axis=0, keepdims=True) / N
        var = jnp.sum(xm * xm, axis=0, keepdims=True) / N - mean * mean
        y = (x - mean) * lax.rsqrt(var + EPS) * gamma_ref[...] + beta_ref[...]
        y = y * 0.5 * (1.0 + lax.erf(y * 0.7071067811865476))
        o_ref[...] = y[:N]


def kernel(feats, nbr_idx, W, gamma, beta):
    feats_pad = jnp.pad(feats, ((0, TPAD - N), (0, 0)))
    nbr_flat = jnp.pad(nbr_idx, ((0, 0), (0, NPAD - N)),
                       constant_values=-1).reshape(-1)
    g = _sc_gather(feats_pad, nbr_flat).reshape(K, NPAD, C)
    out = pl.pallas_call(
        _tc_body,
        grid=(K,),
        in_specs=[
            pl.BlockSpec((1, NPAD, C), lambda k: (k, 0, 0)),
            pl.BlockSpec((1, C, C), lambda k: (k, 0, 0)),
            pl.BlockSpec((1, C), lambda k: (0, 0)),
            pl.BlockSpec((1, C), lambda k: (0, 0)),
        ],
        out_specs=pl.BlockSpec((N, C), lambda k: (0, 0)),
        out_shape=jax.ShapeDtypeStruct((N, C), jnp.float32),
        scratch_shapes=[pltpu.VMEM((NPAD, C), jnp.float32)],
    )(g, W, gamma.reshape(1, C), beta.reshape(1, C))
    return out


# speedup vs baseline: 48.7710x; 48.7710x over previous
import jax
import jax.numpy as jnp
from jax import lax
from jax.experimental import pallas as pl
from jax.experimental.pallas import tpu as pltpu
from jax.experimental.pallas import tpu_sc as plsc

N = 10000
C = 128
K = 27
EPS = 1e-5
JB = 128
NPAD = 10240
NBLK = NPAD // JB
NW = 32
TPAD = 10240
ZROW = N
NBUF = 4


ROWS_PER_TILE = TPAD // 16


def _sc_gather_body(feats_hbm, nbr_hbm, g_hbm, ftab):
    s = lax.axis_index("s")
    pltpu.sync_copy(feats_hbm.at[pl.ds(s * ROWS_PER_TILE, ROWS_PER_TILE)],
                    ftab.at[pl.ds(s * ROWS_PER_TILE, ROWS_PER_TILE)])
    plsc.subcore_barrier()

    def body(i_vmem, o_vmem):
        @pl.loop(0, JB, step=16)
        def _(c):
            v = i_vmem[pl.ds(c, 16)]
            i_vmem[pl.ds(c, 16)] = jnp.where(v >= 0, v, ZROW)

        pltpu.sync_copy(ftab.at[i_vmem], o_vmem)

    pltpu.emit_pipeline(
        body,
        grid=(K * NBLK,),
        in_specs=[pl.BlockSpec((JB,), lambda i: (i,))],
        out_specs=[pl.BlockSpec((JB, C), lambda i: (i, 0))],
        core_axis_name=("c", "s"),
        dimension_semantics=(pltpu.PARALLEL,),
    )(nbr_hbm, g_hbm)


def _sc_gather(feats_pad, nbr_flat):
    mesh = plsc.VectorSubcoreMesh(core_axis_name="c", subcore_axis_name="s")
    f = pl.kernel(
        _sc_gather_body,
        out_type=jax.ShapeDtypeStruct((K * NPAD, C), jnp.float32),
        mesh=mesh,
        scratch_types=[pltpu.VMEM_SHARED((TPAD, C), jnp.float32)],
    )
    return f(feats_pad, nbr_flat)


def _tc_body(g_ref, w_ref, gamma_ref, beta_ref, o_ref, acc_ref):
    k = pl.program_id(0)

    @pl.when(k == 0)
    def _():
        acc_ref[...] = jnp.zeros_like(acc_ref)

    acc_ref[...] += lax.dot_general(
        g_ref[0], w_ref[0], (((1,), (0,)), ((), ())),
        preferred_element_type=jnp.float32)

    @pl.when(k == K - 1)
    def _():
        x = acc_ref[...]
        row = lax.broadcasted_iota(jnp.int32, (NPAD, 1), 0)
        m = (row < N).astype(jnp.float32)
        xm = x * m
        mean = jnp.sum(xm, axis=0, keepdims=True) / N
        var = jnp.sum(xm * xm, axis=0, keepdims=True) / N - mean * mean
        y = (x - mean) * lax.rsqrt(var + EPS) * gamma_ref[...] + beta_ref[...]
        y = y * 0.5 * (1.0 + lax.erf(y * 0.7071067811865476))
        o_ref[...] = y[:N]


def kernel(feats, nbr_idx, W, gamma, beta):
    feats_pad = jnp.pad(feats, ((0, TPAD - N), (0, 0)))
    nbr_flat = jnp.pad(nbr_idx, ((0, 0), (0, NPAD - N)),
                       constant_values=-1).reshape(-1)
    g = _sc_gather(feats_pad, nbr_flat).reshape(K, NPAD, C)
    out = pl.pallas_call(
        _tc_body,
        grid=(K,),
        in_specs=[
            pl.BlockSpec((1, NPAD, C), lambda k: (k, 0, 0)),
            pl.BlockSpec((1, C, C), lambda k: (k, 0, 0)),
            pl.BlockSpec((1, C), lambda k: (0, 0)),
            pl.BlockSpec((1, C), lambda k: (0, 0)),
        ],
        out_specs=pl.BlockSpec((N, C), lambda k: (0, 0)),
        out_shape=jax.ShapeDtypeStruct((N, C), jnp.float32),
        scratch_shapes=[pltpu.VMEM((NPAD, C), jnp.float32)],
    )(g, W, gamma.reshape(1, C), beta.reshape(1, C))
    return out
